# Initial kernel scaffold; baseline (speedup 1.0000x reference)
import jax
import jax.numpy as jnp
from jax.experimental import pallas as pl


def kernel(z1, z2, edge_index):
    raise NotImplementedError("write your pallas kernel here")


# speedup vs baseline: 5.0495x; 5.0495x over previous
import functools

import jax
import jax.numpy as jnp
from jax import lax
from jax.experimental import pallas as pl
from jax.experimental.pallas import tpu as pltpu
from jax.experimental.pallas import tpu_sc as plsc

N_NODES = 10000
N_EDGES = 320000
D = 128
L = 16

NUM_CORES = 2
NUM_SUBCORES = 16
NW = NUM_CORES * NUM_SUBCORES
EPW = N_EDGES // NW
CHUNK = 80
NCHUNK = EPW // CHUNK
NGROUP = CHUNK // L


def _sc_body(src_hbm, dst_hbm, z1_hbm, z2_hbm, out_hbm,
             idx1_v, idx2_v, r1a, r2a, r1b, r2b, o_v, sem_a, sem_b):
    wid = lax.axis_index("s") * NUM_CORES + lax.axis_index("c")
    base = wid * EPW
    lane = lax.iota(jnp.int32, L)

    pltpu.sync_copy(src_hbm.at[pl.ds(base, EPW)], idx1_v)
    pltpu.sync_copy(dst_hbm.at[pl.ds(base, EPW)], idx2_v)

    def fire(ci, r1buf, r2buf, sem):
        loff = ci * CHUNK
        pltpu.async_copy(z1_hbm.at[idx1_v.at[pl.ds(loff, CHUNK)]], r1buf, sem)
        pltpu.async_copy(z2_hbm.at[idx2_v.at[pl.ds(loff, CHUNK)]], r2buf, sem)

    def drain(r1buf, r2buf, sem):
        pltpu.make_async_copy(z1_hbm.at[idx1_v.at[pl.ds(0, CHUNK)]], r1buf, sem).wait()
        pltpu.make_async_copy(z2_hbm.at[idx2_v.at[pl.ds(0, CHUNK)]], r2buf, sem).wait()

    def compute(ci, r1buf, r2buf):
        loff = ci * CHUNK

        def group_body(g, c):
            vec = jnp.zeros((L,), jnp.float32)
            for k in range(L):
                e = g * L + k
                acc = r1buf[e, pl.ds(0, L)] * r2buf[e, pl.ds(0, L)]
                for j in range(1, D // L):
                    acc = acc + r1buf[e, pl.ds(j * L, L)] * r2buf[e, pl.ds(j * L, L)]
                for s in (1, 2, 4, 8):
                    acc = acc + jnp.take_along_axis(
                        acc, lane ^ s, axis=0, mode="promise_in_bounds")
                vec = jnp.where(lane == k, acc, vec)
            o_v[pl.ds(loff + g * L, L)] = 1.0 / (1.0 + jnp.exp(-vec))
            return c

        lax.fori_loop(0, NGROUP, group_body, 0)

    fire(0, r1a, r2a, sem_a)

    def pair_body(i, c):
        fire(2 * i + 1, r1b, r2b, sem_b)
        drain(r1a, r2a, sem_a)
        compute(2 * i, r1a, r2a)
        fire(2 * i + 2, r1a, r2a, sem_a)
        drain(r1b, r2b, sem_b)
        compute(2 * i + 1, r1b, r2b)
        return c

    lax.fori_loop(0, (NCHUNK - 1) // 2, pair_body, 0)
    drain(r1a, r2a, sem_a)
    compute(NCHUNK - 1, r1a, r2a)

    pltpu.sync_copy(o_v, out_hbm.at[pl.ds(base, EPW)])


@jax.jit
def _run(src, dst, z1, z2):
    mesh = plsc.VectorSubcoreMesh(core_axis_name="c", subcore_axis_name="s")
    f = pl.kernel(
        _sc_body,
        out_type=jax.ShapeDtypeStruct((N_EDGES,), jnp.float32),
        mesh=mesh,
        scratch_types=[
            pltpu.VMEM((EPW,), jnp.int32),
            pltpu.VMEM((EPW,), jnp.int32),
            pltpu.VMEM((CHUNK, D), jnp.float32),
            pltpu.VMEM((CHUNK, D), jnp.float32),
            pltpu.VMEM((CHUNK, D), jnp.float32),
            pltpu.VMEM((CHUNK, D), jnp.float32),
            pltpu.VMEM((EPW,), jnp.float32),
            pltpu.SemaphoreType.DMA,
            pltpu.SemaphoreType.DMA,
        ],
    )
    return f(src, dst, z1, z2)


def kernel(z1, z2, edge_index):
    src = edge_index[0].astype(jnp.int32)
    dst = edge_index[1].astype(jnp.int32)
    return _run(src, dst, z1, z2)
